# baseline (speedup 1.0000x reference)
import math

import jax
import jax.numpy as jnp
from jax.experimental import pallas as pl
from jax.experimental.pallas import tpu as pltpu

D = 1024
ROWS = 2 * 4096 - 1
BATCH = 4
BLOCK = 256
NBLK = (ROWS + BLOCK - 1) // BLOCK
LAST_BASE = (NBLK - 1) * BLOCK
LAST_ROWS = ROWS - LAST_BASE
NBUF = 6
LOOKAHEAD = 3
SCALE = math.sqrt(D)


def _body(w_hbm, o_hbm, b0, b1, b2, b3, b4, b5, blast, sin_ref, sout_ref):
    bufs = [b0, b1, b2, b3, b4, b5]

    def buf(k):
        return blast if k == NBLK - 1 else bufs[k % NBUF]

    def blk(k):
        return (k * BLOCK, BLOCK) if k < NBLK - 1 else (LAST_BASE, LAST_ROWS)

    def issue_gather(k):
        base, nrows = blk(k)
        h = pltpu.make_async_copy(
            w_hbm.at[pl.ds(base, nrows), :], buf(k), sin_ref.at[k % NBUF])
        h.start()
        return h

    def issue_scatters(k):
        base, nrows = blk(k)
        hs = [
            pltpu.make_async_copy(
                buf(k), o_hbm.at[b, pl.ds(base, nrows), :],
                sout_ref.at[k % NBUF])
            for b in range(BATCH)
        ]
        for h in hs:
            h.start()
        return hs

    gathers = {k: issue_gather(k) for k in range(LOOKAHEAD)}
    scatters = {}
    for g in range(NBLK):
        if g - LOOKAHEAD in scatters:
            for h in scatters.pop(g - LOOKAHEAD):
                h.wait()
        if g + LOOKAHEAD < NBLK:
            gathers[g + LOOKAHEAD] = issue_gather(g + LOOKAHEAD)
        gathers.pop(g).wait()

        b = buf(g)
        b[...] = b[...] * SCALE

        scatters[g] = issue_scatters(g)

    for hs in scatters.values():
        for h in hs:
            h.wait()


def _tc_embed(weights):
    return pl.pallas_call(
        _body,
        in_specs=[pl.BlockSpec(memory_space=pltpu.HBM)],
        out_specs=pl.BlockSpec(memory_space=pltpu.HBM),
        out_shape=jax.ShapeDtypeStruct((BATCH, ROWS, D), jnp.float32),
        scratch_shapes=(
            [pltpu.VMEM((BLOCK, D), jnp.float32) for _ in range(NBUF)]
            + [pltpu.VMEM((LAST_ROWS, D), jnp.float32)]
            + [pltpu.SemaphoreType.DMA((NBUF,)),
               pltpu.SemaphoreType.DMA((NBUF,))]
        ),
    )(weights)


def kernel(input, weights):
    del input
    return _tc_embed(weights)


# speedup vs baseline: 1.0000x; 1.0000x over previous
import math

import jax
import jax.numpy as jnp
from jax.experimental import pallas as pl
from jax.experimental.pallas import tpu as pltpu

D = 1024
ROWS = 2 * 4096 - 1
BATCH = 4
BLOCK = 256
NBLK = (ROWS + BLOCK - 1) // BLOCK
LAST_BASE = (NBLK - 1) * BLOCK
LAST_ROWS = ROWS - LAST_BASE
NBUF = 6
LOOKAHEAD = 3
SCALE = math.sqrt(D)


def _body(w_hbm, o_hbm, b0, b1, b2, b3, b4, b5, blast, sin_ref,
          so0, so1, so2, so3):
    bufs = [b0, b1, b2, b3, b4, b5]
    souts = [so0, so1, so2, so3]

    def buf(k):
        return blast if k == NBLK - 1 else bufs[k % NBUF]

    def blk(k):
        return (k * BLOCK, BLOCK) if k < NBLK - 1 else (LAST_BASE, LAST_ROWS)

    def issue_gather(k):
        base, nrows = blk(k)
        h = pltpu.make_async_copy(
            w_hbm.at[pl.ds(base, nrows), :], buf(k), sin_ref.at[k % NBUF])
        h.start()
        return h

    def issue_scatters(k):
        base, nrows = blk(k)
        hs = [
            pltpu.make_async_copy(
                buf(k), o_hbm.at[b, pl.ds(base, nrows), :],
                souts[b].at[k % NBUF])
            for b in range(BATCH)
        ]
        for h in hs:
            h.start()
        return hs

    gathers = {k: issue_gather(k) for k in range(LOOKAHEAD)}
    scatters = {}
    for g in range(NBLK):
        if g - LOOKAHEAD in scatters:
            for h in scatters.pop(g - LOOKAHEAD):
                h.wait()
        if g + LOOKAHEAD < NBLK:
            gathers[g + LOOKAHEAD] = issue_gather(g + LOOKAHEAD)
        gathers.pop(g).wait()

        b = buf(g)
        b[...] = b[...] * SCALE

        scatters[g] = issue_scatters(g)

    for hs in scatters.values():
        for h in hs:
            h.wait()


def _tc_embed(weights):
    return pl.pallas_call(
        _body,
        in_specs=[pl.BlockSpec(memory_space=pltpu.HBM)],
        out_specs=pl.BlockSpec(memory_space=pltpu.HBM),
        out_shape=jax.ShapeDtypeStruct((BATCH, ROWS, D), jnp.float32),
        scratch_shapes=(
            [pltpu.VMEM((BLOCK, D), jnp.float32) for _ in range(NBUF)]
            + [pltpu.VMEM((LAST_ROWS, D), jnp.float32)]
            + [pltpu.SemaphoreType.DMA((NBUF,)) for _ in range(5)]
        ),
    )(weights)


def kernel(input, weights):
    del input
    return _tc_embed(weights)


# speedup vs baseline: 1.0127x; 1.0126x over previous
import math

import jax
import jax.numpy as jnp
from jax.experimental import pallas as pl
from jax.experimental.pallas import tpu as pltpu

D = 1024
ROWS = 2 * 4096 - 1
BATCH = 4
BLOCK_ROWS = 1024
GRID = (ROWS + BLOCK_ROWS - 1) // BLOCK_ROWS
SCALE = math.sqrt(D)


def _body(w_ref, o_ref):
    scaled = w_ref[...] * SCALE
    o_ref[...] = jnp.broadcast_to(scaled[None, :, :], (BATCH,) + scaled.shape)


def _tc_embed(weights):
    return pl.pallas_call(
        _body,
        grid=(GRID,),
        in_specs=[pl.BlockSpec((BLOCK_ROWS, D), lambda i: (i, 0))],
        out_specs=pl.BlockSpec((BATCH, BLOCK_ROWS, D), lambda i: (0, i, 0)),
        out_shape=jax.ShapeDtypeStruct((BATCH, ROWS, D), jnp.float32),
        compiler_params=pltpu.CompilerParams(
            dimension_semantics=("arbitrary",),
        ),
    )(weights)


def kernel(input, weights):
    del input
    return _tc_embed(weights)
